# baseline (speedup 1.0000x reference)
import functools

import jax
import jax.numpy as jnp
from jax import lax
from jax.experimental import pallas as pl
from jax.experimental.pallas import tpu as pltpu
from jax.experimental.pallas import tpu_sc as plsc

DT = 0.01
ITERATION = 3

NC = 2
NS = 16
CHUNK = 512
GROUPS = CHUNK // 16


def _ceil_to(x, m):
    return (x + m - 1) // m * m


def _body(NVp, VT, ET, NCH, EPALL,
          VpF, VelpF, FpF, MWCpF, i0f, i1f, d0f,
          VoF, VeloF, Lout, REC,
          *scr):
    (PX, PY, PZ, AX, AY, AZ, W, Q) = scr[0:8]
    setA = scr[8:24]
    setB = scr[24:40]
    (sa, sb, sc_, sd, pxs, pys, pzs, ws, qs) = scr[40:49]
    (sem, seml, sems) = scr[49:52]
    c = lax.axis_index("c")
    s = lax.axis_index("s")
    v0 = s * VT
    e0 = s * ET

    iota = lax.iota(jnp.int32, 16)
    iota3 = iota * 3
    MAGIC = jnp.full((16,), 0x5F3759DF, jnp.int32)
    fzero = jnp.zeros((16,), jnp.float32)

    def predict_chunk(vb, n):
        fb = (c * NVp + vb) * 3
        pltpu.sync_copy(VpF.at[pl.ds(fb, n * 3)], sa.at[pl.ds(0, n * 3)])
        pltpu.sync_copy(VelpF.at[pl.ds(fb, n * 3)], sb.at[pl.ds(0, n * 3)])
        pltpu.sync_copy(FpF.at[pl.ds(fb, n * 3)], sc_.at[pl.ds(0, n * 3)])
        pltpu.sync_copy(MWCpF.at[pl.ds(fb, n * 3)], sd.at[pl.ds(0, n * 3)])

        def pg(g, _):
            base = 48 * g
            dg = pl.ds(g * 16, 16)
            mv = plsc.load_gather(sd, [iota3 + base])
            k = (DT * DT) / mv
            ws[dg] = plsc.load_gather(sd, [iota3 + (base + 1)])
            qs[dg] = plsc.load_gather(sd, [iota3 + (base + 2)])
            for col, dst in ((0, pxs), (1, pys), (2, pzs)):
                x = plsc.load_gather(sa, [iota3 + (base + col)])
                v = plsc.load_gather(sb, [iota3 + (base + col)])
                f = plsc.load_gather(sc_, [iota3 + (base + col)])
                dst[dg] = x + DT * v + k * f
            return 0
        lax.fori_loop(0, n // 16, pg, 0)
        for stg, p_pl, a_pl in ((pxs, PX, AX), (pys, PY, AY), (pzs, PZ, AZ)):
            pltpu.sync_copy(stg.at[pl.ds(0, n)], p_pl.at[pl.ds(vb, n)])
            pltpu.sync_copy(stg.at[pl.ds(0, n)], a_pl.at[pl.ds(vb, n)])
        pltpu.sync_copy(ws.at[pl.ds(0, n)], W.at[pl.ds(vb, n)])
        pltpu.sync_copy(qs.at[pl.ds(0, n)], Q.at[pl.ds(vb, n)])

    def pred_body(i, _):
        predict_chunk(v0 + i * 512, 512)
        return 0
    lax.fori_loop(0, VT // 512, pred_body, 0)
    if VT % 512:
        predict_chunk(v0 + VT - VT % 512, VT % 512)
    plsc.subcore_barrier()

    def load_chunk(eb, bufs):
        eg = c * EPALL + e0 + eb
        hs = [pltpu.async_copy(i0f.at[pl.ds(e0 + eb, CHUNK)], bufs[1], seml),
              pltpu.async_copy(i1f.at[pl.ds(e0 + eb, CHUNK)], bufs[2], seml),
              pltpu.async_copy(REC.at[pl.ds(eg * 5, 5 * CHUNK)], bufs[3],
                               seml),
              pltpu.async_copy(Lout.at[pl.ds(eg, CHUNK)], bufs[0], seml)]
        return hs

    def fire_gathers(bufs):
        (idx0, idx1) = bufs[1:3]
        (x0b, y0b, z0b, x1b, y1b, z1b) = bufs[4:10]
        hs = []
        for j in range(CHUNK // 128):
            dj = pl.ds(j * 128, 128)
            r0 = idx0.at[dj]
            r1 = idx1.at[dj]
            for plane, dst in ((PX, x0b), (PY, y0b), (PZ, z0b)):
                hs.append(pltpu.async_copy(plane.at[r0], dst.at[dj], sem))
            for plane, dst in ((PX, x1b), (PY, y1b), (PZ, z1b)):
                hs.append(pltpu.async_copy(plane.at[r1], dst.at[dj], sem))
        return hs

    def compute_chunk(eb, bufs, first):
        (lbuf, rec) = (bufs[0], bufs[3])
        (x0b, y0b, z0b, x1b, y1b, z1b) = bufs[4:10]
        (ux0, uy0, uz0, ux1, uy1, uz1) = bufs[10:16]

        def one(g16):
            dg = pl.ds(g16, 16)
            x0 = x0b[dg]
            y0 = y0b[dg]
            z0 = z0b[dg]
            x1 = x1b[dg]
            y1 = y1b[dg]
            z1 = z1b[dg]
            dx = x0 - x1
            dy = y0 - y1
            dz = z0 - z1
            d2 = dx * dx + dy * dy + dz * dz
            ib = MAGIC - lax.shift_right_arithmetic(
                plsc.bitcast(d2, jnp.int32), 1)
            r = plsc.bitcast(ib, jnp.float32)
            r = r * (1.5 - 0.5 * d2 * r * r)
            r = r * (1.5 - 0.5 * d2 * r * r)
            r = r * (1.5 - 0.5 * d2 * r * r)
            d0v = rec[dg]
            w0 = rec[pl.ds(CHUNK + g16, 16)]
            w1 = rec[pl.ds(2 * CHUNK + g16, 16)]
            A = rec[pl.ds(3 * CHUNK + g16, 16)]
            rD = rec[pl.ds(4 * CHUNK + g16, 16)]
            Cc = d2 * r - d0v
            Lv = jnp.where(first, 0.0, lbuf[dg])
            Ld = fzero - (Cc + A * Lv) * rD
            lbuf[dg] = Lv + Ld
            t = Ld * r
            g0 = w0 * t
            g1 = fzero - w1 * t
            ux0[dg] = g0 * dx
            uy0[dg] = g0 * dy
            uz0[dg] = g0 * dz
            ux1[dg] = g1 * dx
            uy1[dg] = g1 * dy
            uz1[dg] = g1 * dz

        def grp(g, _):
            one(g * 32)
            one(g * 32 + 16)
            return 0
        lax.fori_loop(0, GROUPS // 2, grp, 0)
        pltpu.sync_copy(lbuf, Lout.at[pl.ds(c * EPALL + e0 + eb, CHUNK)])

    def fire_scatters(bufs):
        (idx0, idx1) = bufs[1:3]
        (ux0, uy0, uz0, ux1, uy1, uz1) = bufs[10:16]
        hs = []
        for j in range(CHUNK // 128):
            dj = pl.ds(j * 128, 128)
            r0 = idx0.at[dj]
            r1 = idx1.at[dj]
            for usrc, plane in ((ux0, AX), (uy0, AY), (uz0, AZ)):
                hs.append(pltpu.async_copy(usrc.at[dj], plane.at[r0],
                                           sems, add=True))
            for usrc, plane in ((ux1, AX), (uy1, AY), (uz1, AZ)):
                hs.append(pltpu.async_copy(usrc.at[dj], plane.at[r1],
                                           sems, add=True))
        return hs

    def pre_body(ci, _):
        eb = ci * CHUNK
        (idx0, idx1, rec) = setA[1:4]
        (q0b, q1b) = setA[4:6]
        hl = [pltpu.async_copy(i0f.at[pl.ds(e0 + eb, CHUNK)], idx0, seml),
              pltpu.async_copy(i1f.at[pl.ds(e0 + eb, CHUNK)], idx1, seml),
              pltpu.async_copy(d0f.at[pl.ds(e0 + eb, CHUNK)],
                               rec.at[pl.ds(0, CHUNK)], seml)]
        for h in hl:
            h.wait()
        hs = []
        for j in range(CHUNK // 128):
            dj = pl.ds(j * 128, 128)
            r0 = idx0.at[dj]
            r1 = idx1.at[dj]
            hs.append(pltpu.async_copy(
                W.at[r0], rec.at[pl.ds(CHUNK + j * 128, 128)], sem))
            hs.append(pltpu.async_copy(
                W.at[r1], rec.at[pl.ds(2 * CHUNK + j * 128, 128)], sem))
            hs.append(pltpu.async_copy(Q.at[r0], q0b.at[dj], sem))
            hs.append(pltpu.async_copy(Q.at[r1], q1b.at[dj], sem))
        for h in hs:
            h.wait()

        def pg(g, _):
            g16 = g * 16
            dg = pl.ds(g16, 16)
            A = 0.5 * (q0b[dg] + q1b[dg])
            w0 = rec[pl.ds(CHUNK + g16, 16)]
            w1 = rec[pl.ds(2 * CHUNK + g16, 16)]
            rec[pl.ds(3 * CHUNK + g16, 16)] = A
            rec[pl.ds(4 * CHUNK + g16, 16)] = 1.0 / (w0 + w1 + A)
            return 0
        lax.fori_loop(0, GROUPS, pg, 0)
        eg = c * EPALL + e0 + eb
        pltpu.sync_copy(rec, REC.at[pl.ds(eg * 5, 5 * CHUNK)])
        return 0
    lax.fori_loop(0, NCH, pre_body, 0)

    def iter_body(itv, _):
        first = itv == 0

        def pair_body(ci, _):
            ebA = ci * (2 * CHUNK)
            ebB = ebA + CHUNK
            hlA = load_chunk(ebA, setA)
            hlB = load_chunk(ebB, setB)
            hlA[0].wait()
            hlA[1].wait()
            hgA = fire_gathers(setA)
            hlB[0].wait()
            hlB[1].wait()
            hgB = fire_gathers(setB)
            for h in hgA + hlA[2:]:
                h.wait()
            compute_chunk(ebA, setA, first)
            hsA = fire_scatters(setA)
            for h in hgB + hlB[2:]:
                h.wait()
            compute_chunk(ebB, setB, first)
            hsB = fire_scatters(setB)
            for h in hsA:
                h.wait()
            for h in hsB:
                h.wait()
            return 0
        lax.fori_loop(0, NCH // 2, pair_body, 0)
        plsc.subcore_barrier()
        def copy_body(i2, _):
            vb = v0 + i2 * 512
            for stg, p_pl, a_pl in ((pxs, PX, AX), (pys, PY, AY),
                                    (pzs, PZ, AZ)):
                pltpu.sync_copy(a_pl.at[pl.ds(vb, 512)], stg.at[pl.ds(0, 512)])
                pltpu.sync_copy(stg.at[pl.ds(0, 512)], p_pl.at[pl.ds(vb, 512)])
            return 0
        lax.fori_loop(0, VT // 512, copy_body, 0)
        if VT % 512:
            vb = v0 + VT - VT % 512
            n = VT % 512
            for stg, p_pl, a_pl in ((pxs, PX, AX), (pys, PY, AY),
                                    (pzs, PZ, AZ)):
                pltpu.sync_copy(a_pl.at[pl.ds(vb, n)], stg.at[pl.ds(0, n)])
                pltpu.sync_copy(stg.at[pl.ds(0, n)], p_pl.at[pl.ds(vb, n)])
        plsc.subcore_barrier()
        return 0
    lax.fori_loop(0, ITERATION, iter_body, 0)

    def final_chunk(vb, n):
        fb = (c * NVp + vb) * 3
        pltpu.sync_copy(AX.at[pl.ds(vb, n)], pxs.at[pl.ds(0, n)])
        pltpu.sync_copy(AY.at[pl.ds(vb, n)], pys.at[pl.ds(0, n)])
        pltpu.sync_copy(AZ.at[pl.ds(vb, n)], pzs.at[pl.ds(0, n)])
        pltpu.sync_copy(VpF.at[pl.ds(fb, n * 3)], sa.at[pl.ds(0, n * 3)])

        def fg(g, _):
            base = 48 * g
            dg = pl.ds(g * 16, 16)
            for col, stg in ((0, pxs), (1, pys), (2, pzs)):
                px = stg[dg]
                xv = plsc.load_gather(sa, [iota3 + (base + col)])
                plsc.store_scatter(sb, [iota3 + (base + col)], px)
                plsc.store_scatter(sc_, [iota3 + (base + col)],
                                   (px - xv) * (1.0 / DT))
            return 0
        lax.fori_loop(0, n // 16, fg, 0)
        pltpu.sync_copy(sb.at[pl.ds(0, n * 3)], VoF.at[pl.ds(fb, n * 3)])
        pltpu.sync_copy(sc_.at[pl.ds(0, n * 3)], VeloF.at[pl.ds(fb, n * 3)])

    def fin_body(i, _):
        final_chunk(v0 + i * 512, 512)
        return 0
    lax.fori_loop(0, VT // 512, fin_body, 0)
    if VT % 512:
        final_chunk(v0 + VT - VT % 512, VT % 512)


def kernel(V, V_velocity, V_w, V_mass, V_force, V_compliance, C_dist, C_init_d):
    B, NV, _ = V.shape
    E = C_dist.shape[0]
    assert B == NC
    VT = _ceil_to(-(-NV // NS), 16)
    NVp = VT * NS
    ET = _ceil_to(-(-E // NS), CHUNK)
    Ep = ET * NS
    NCH = ET // CHUNK

    pv = NVp - NV
    f32 = jnp.float32
    i32 = jnp.int32
    VpF = jnp.concatenate([V, jnp.zeros((B, pv, 3), f32)], axis=1).reshape(-1)
    VelpF = jnp.concatenate([V_velocity, jnp.zeros((B, pv, 3), f32)],
                            axis=1).reshape(-1)
    FpF = jnp.concatenate([V_force, jnp.zeros((B, pv, 3), f32)],
                          axis=1).reshape(-1)
    MWC = jnp.concatenate([V_mass, V_w, V_compliance], axis=-1)
    MWCpF = jnp.concatenate(
        [MWC, jnp.broadcast_to(jnp.array([1.0, 1.0, 0.0], f32), (B, pv, 3))],
        axis=1).reshape(-1)

    pe = Ep - E
    i0f = jnp.concatenate([C_dist[:, 0].astype(i32),
                           jnp.full((pe,), NV, i32)])
    i1f = jnp.concatenate([C_dist[:, 1].astype(i32),
                           jnp.full((pe,), NV, i32)])
    d0f = jnp.concatenate([C_init_d[:, 0], jnp.zeros((pe,), f32)])

    mesh = plsc.VectorSubcoreMesh(core_axis_name="c", subcore_axis_name="s")
    body = functools.partial(_body, NVp, VT, ET, NCH, Ep)
    run = pl.kernel(
        body,
        out_type=(jax.ShapeDtypeStruct((B * NVp * 3,), f32),
                  jax.ShapeDtypeStruct((B * NVp * 3,), f32),
                  jax.ShapeDtypeStruct((NC * Ep,), f32),
                  jax.ShapeDtypeStruct((NC * Ep * 5,), f32)),
        mesh=mesh,
        compiler_params=pltpu.CompilerParams(use_tc_tiling_on_sc=False,
                                             needs_layout_passes=False),
        scratch_types=(
            [pltpu.VMEM_SHARED((NVp,), f32) for _ in range(8)]
            + ([pltpu.VMEM((CHUNK,), f32),
                pltpu.VMEM((CHUNK,), i32),
                pltpu.VMEM((CHUNK,), i32),
                pltpu.VMEM((5 * CHUNK,), f32)]
               + [pltpu.VMEM((CHUNK,), f32) for _ in range(12)]) * 2
            + [pltpu.VMEM((1536,), f32) for _ in range(4)]
            + [pltpu.VMEM((512,), f32) for _ in range(5)]
            + [pltpu.SemaphoreType.DMA for _ in range(3)]
        ),
    )
    VoF, VeloF, *_ = run(VpF, VelpF, FpF, MWCpF, i0f, i1f, d0f)
    Vo = VoF.reshape(B, NVp, 3)[:, :NV, :]
    Velo = VeloF.reshape(B, NVp, 3)[:, :NV, :]
    return Vo, Velo


# speedup vs baseline: 1.2087x; 1.2087x over previous
import functools

import jax
import jax.numpy as jnp
from jax import lax
from jax.experimental import pallas as pl
from jax.experimental.pallas import tpu as pltpu
from jax.experimental.pallas import tpu_sc as plsc

DT = 0.01
ITERATION = 3

NC = 2
NS = 16
CHUNK = 1024
GROUPS = CHUNK // 16


def _ceil_to(x, m):
    return (x + m - 1) // m * m


def _body(NVp, VT, ET, NCH, EPALL,
          VpF, VelpF, FpF, MWCpF, i0f, i1f, d0f,
          VoF, VeloF, Lout, W0e, W1e, Ae_, rDe,
          *scr):
    (PX, PY, PZ, AX, AY, AZ, W, Q) = scr[0:8]
    setA = scr[8:29]
    setB = scr[29:50]
    (sa, sb, sc_, sd, pxs, pys, pzs, ws, qs) = scr[50:59]
    (sem, seml, sems) = scr[59:62]
    c = lax.axis_index("c")
    s = lax.axis_index("s")
    v0 = s * VT
    e0 = s * ET

    iota = lax.iota(jnp.int32, 16)
    iota3 = iota * 3
    MAGIC = jnp.full((16,), 0x5F3759DF, jnp.int32)
    fzero = jnp.zeros((16,), jnp.float32)

    def predict_chunk(vb, n):
        fb = (c * NVp + vb) * 3
        pltpu.sync_copy(VpF.at[pl.ds(fb, n * 3)], sa.at[pl.ds(0, n * 3)])
        pltpu.sync_copy(VelpF.at[pl.ds(fb, n * 3)], sb.at[pl.ds(0, n * 3)])
        pltpu.sync_copy(FpF.at[pl.ds(fb, n * 3)], sc_.at[pl.ds(0, n * 3)])
        pltpu.sync_copy(MWCpF.at[pl.ds(fb, n * 3)], sd.at[pl.ds(0, n * 3)])

        def pg(g, _):
            base = 48 * g
            dg = pl.ds(g * 16, 16)
            mv = plsc.load_gather(sd, [iota3 + base])
            k = (DT * DT) / mv
            ws[dg] = plsc.load_gather(sd, [iota3 + (base + 1)])
            qs[dg] = plsc.load_gather(sd, [iota3 + (base + 2)])
            for col, dst in ((0, pxs), (1, pys), (2, pzs)):
                x = plsc.load_gather(sa, [iota3 + (base + col)])
                v = plsc.load_gather(sb, [iota3 + (base + col)])
                f = plsc.load_gather(sc_, [iota3 + (base + col)])
                dst[dg] = x + DT * v + k * f
            return 0
        lax.fori_loop(0, n // 16, pg, 0)
        for stg, p_pl, a_pl in ((pxs, PX, AX), (pys, PY, AY), (pzs, PZ, AZ)):
            pltpu.sync_copy(stg.at[pl.ds(0, n)], p_pl.at[pl.ds(vb, n)])
            pltpu.sync_copy(stg.at[pl.ds(0, n)], a_pl.at[pl.ds(vb, n)])
        pltpu.sync_copy(ws.at[pl.ds(0, n)], W.at[pl.ds(vb, n)])
        pltpu.sync_copy(qs.at[pl.ds(0, n)], Q.at[pl.ds(vb, n)])

    def pred_body(i, _):
        predict_chunk(v0 + i * 512, 512)
        return 0
    lax.fori_loop(0, VT // 512, pred_body, 0)
    if VT % 512:
        predict_chunk(v0 + VT - VT % 512, VT % 512)
    plsc.subcore_barrier()

    def load_chunk(eb, bufs):
        (lbuf, ibuf, d0c) = bufs[0:3]
        hs = [pltpu.async_copy(i0f.at[pl.ds(e0 + eb, CHUNK)],
                               ibuf.at[pl.ds(0, CHUNK)], seml),
              pltpu.async_copy(i1f.at[pl.ds(e0 + eb, CHUNK)],
                               ibuf.at[pl.ds(CHUNK, CHUNK)], seml),
              pltpu.async_copy(d0f.at[pl.ds(e0 + eb, CHUNK)], d0c, seml),
              pltpu.async_copy(Lout.at[pl.ds(c * EPALL + e0 + eb, CHUNK)],
                               lbuf, seml),
              pltpu.async_copy(W0e.at[pl.ds(c * EPALL + e0 + eb, CHUNK)],
                               bufs[8], seml),
              pltpu.async_copy(W1e.at[pl.ds(c * EPALL + e0 + eb, CHUNK)],
                               bufs[13], seml),
              pltpu.async_copy(Ae_.at[pl.ds(c * EPALL + e0 + eb, CHUNK)],
                               bufs[9], seml),
              pltpu.async_copy(rDe.at[pl.ds(c * EPALL + e0 + eb, CHUNK)],
                               bufs[14], seml)]
        return hs

    def stage_idx(bufs):
        (ibuf, idx0, idx1) = (bufs[1], bufs[3], bufs[4])

        def st(t, _):
            rowi = lax.shift_right_logical(jnp.full((16,), 0, jnp.int32) + t, 3)
            coli = iota + lax.shift_left(
                lax.bitwise_and(t, 7), 4)
            plsc.store_scatter(idx0, [rowi, coli], ibuf[pl.ds(t * 16, 16)])
            plsc.store_scatter(idx1, [rowi, coli],
                               ibuf[pl.ds(CHUNK + t * 16, 16)])
            return 0
        lax.fori_loop(0, GROUPS, st, 0)

    def fire_gathers(bufs):
        (idx0, idx1) = bufs[3:5]
        (x0b, y0b, z0b, w0b, q0b, x1b, y1b, z1b, w1b, q1b) = bufs[5:15]
        hs = []
        for j in range(CHUNK // 128):
            dj = pl.ds(j * 128, 128)
            r0 = idx0.at[j]
            r1 = idx1.at[j]
            for plane, dst in ((PX, x0b), (PY, y0b), (PZ, z0b)):
                hs.append(pltpu.async_copy(plane.at[r0], dst.at[dj], sem))
            for plane, dst in ((PX, x1b), (PY, y1b), (PZ, z1b)):
                hs.append(pltpu.async_copy(plane.at[r1], dst.at[dj], sem))
        return hs

    def compute_chunk(eb, bufs, first):
        (lbuf, d0c) = (bufs[0], bufs[2])
        (x0b, y0b, z0b, w0b, q0b, x1b, y1b, z1b, w1b, q1b) = bufs[5:15]
        (ux0, uy0, uz0, ux1, uy1, uz1) = bufs[15:21]

        def one(dg):
            x0 = x0b[dg]
            y0 = y0b[dg]
            z0 = z0b[dg]
            w0 = w0b[dg]
            q0 = q0b[dg]
            x1 = x1b[dg]
            y1 = y1b[dg]
            z1 = z1b[dg]
            w1 = w1b[dg]
            q1 = q1b[dg]
            dx = x0 - x1
            dy = y0 - y1
            dz = z0 - z1
            d2 = dx * dx + dy * dy + dz * dz
            ib = MAGIC - lax.shift_right_arithmetic(
                plsc.bitcast(d2, jnp.int32), 1)
            r = plsc.bitcast(ib, jnp.float32)
            r = r * (1.5 - 0.5 * d2 * r * r)
            r = r * (1.5 - 0.5 * d2 * r * r)
            r = r * (1.5 - 0.5 * d2 * r * r)
            d0v = d0c[dg]
            A = q0
            rD = q1
            Cc = d2 * r - d0v
            Lv = jnp.where(first, 0.0, lbuf[dg])
            Ld = fzero - (Cc + A * Lv) * rD
            lbuf[dg] = Lv + Ld
            t = Ld * r
            g0 = w0 * t
            g1 = fzero - w1 * t
            ux0[dg] = g0 * dx
            uy0[dg] = g0 * dy
            uz0[dg] = g0 * dz
            ux1[dg] = g1 * dx
            uy1[dg] = g1 * dy
            uz1[dg] = g1 * dz

        def grp(g, _):
            one(pl.ds(g * 32, 16))
            one(pl.ds(g * 32 + 16, 16))
            return 0
        lax.fori_loop(0, GROUPS // 2, grp, 0)
        pltpu.sync_copy(bufs[0], Lout.at[pl.ds(c * EPALL + e0 + eb, CHUNK)])

    def fire_scatters(bufs):
        (idx0, idx1) = bufs[3:5]
        (ux0, uy0, uz0, ux1, uy1, uz1) = bufs[15:21]
        hs = []
        for j in range(CHUNK // 128):
            dj = pl.ds(j * 128, 128)
            r0 = idx0.at[j]
            r1 = idx1.at[j]
            for usrc, plane in ((ux0, AX), (uy0, AY), (uz0, AZ)):
                hs.append(pltpu.async_copy(usrc.at[dj], plane.at[r0],
                                           sems, add=True))
            for usrc, plane in ((ux1, AX), (uy1, AY), (uz1, AZ)):
                hs.append(pltpu.async_copy(usrc.at[dj], plane.at[r1],
                                           sems, add=True))
        return hs

    def pre_body(ci, _):
        eb = ci * CHUNK
        (lbuf, ibuf, d0c, idx0, idx1) = setA[0:5]
        (x0b, y0b, z0b, w0b, q0b, x1b, y1b, z1b, w1b, q1b) = setA[5:15]
        hl = [pltpu.async_copy(i0f.at[pl.ds(e0 + eb, CHUNK)],
                               ibuf.at[pl.ds(0, CHUNK)], seml),
              pltpu.async_copy(i1f.at[pl.ds(e0 + eb, CHUNK)],
                               ibuf.at[pl.ds(CHUNK, CHUNK)], seml)]
        for h in hl:
            h.wait()
        stage_idx(setA)
        hs = []
        for j in range(CHUNK // 128):
            dj = pl.ds(j * 128, 128)
            for plane, dst in ((W, w0b), (Q, q0b)):
                hs.append(pltpu.async_copy(plane.at[idx0.at[j]],
                                           dst.at[dj], sem))
            for plane, dst in ((W, w1b), (Q, q1b)):
                hs.append(pltpu.async_copy(plane.at[idx1.at[j]],
                                           dst.at[dj], sem))
        for h in hs:
            h.wait()

        def pg(g, _):
            dg = pl.ds(g * 16, 16)
            A = 0.5 * (q0b[dg] + q1b[dg])
            rD = 1.0 / (w0b[dg] + w1b[dg] + A)
            x0b[dg] = A
            y0b[dg] = rD
            return 0
        lax.fori_loop(0, GROUPS, pg, 0)
        base = pl.ds(c * EPALL + e0 + eb, CHUNK)
        pltpu.sync_copy(w0b, W0e.at[base])
        pltpu.sync_copy(w1b, W1e.at[base])
        pltpu.sync_copy(x0b, Ae_.at[base])
        pltpu.sync_copy(y0b, rDe.at[base])
        return 0
    lax.fori_loop(0, NCH, pre_body, 0)

    def iter_body(itv, _):
        first = itv == 0

        def pair_body(ci, _):
            ebA = ci * (2 * CHUNK)
            ebB = ebA + CHUNK
            hlA = load_chunk(ebA, setA)
            hlB = load_chunk(ebB, setB)
            for h in hlA:
                h.wait()
            stage_idx(setA)
            hgA = fire_gathers(setA)
            for h in hlB:
                h.wait()
            stage_idx(setB)
            hgB = fire_gathers(setB)
            for h in hgA:
                h.wait()
            compute_chunk(ebA, setA, first)
            hsA = fire_scatters(setA)
            for h in hgB:
                h.wait()
            compute_chunk(ebB, setB, first)
            hsB = fire_scatters(setB)
            for h in hsA:
                h.wait()
            for h in hsB:
                h.wait()
            return 0
        lax.fori_loop(0, NCH // 2, pair_body, 0)
        plsc.subcore_barrier()
        def copy_body(i2, _):
            vb = v0 + i2 * 512
            for stg, p_pl, a_pl in ((pxs, PX, AX), (pys, PY, AY),
                                    (pzs, PZ, AZ)):
                pltpu.sync_copy(a_pl.at[pl.ds(vb, 512)], stg.at[pl.ds(0, 512)])
                pltpu.sync_copy(stg.at[pl.ds(0, 512)], p_pl.at[pl.ds(vb, 512)])
            return 0
        lax.fori_loop(0, VT // 512, copy_body, 0)
        if VT % 512:
            vb = v0 + VT - VT % 512
            n = VT % 512
            for stg, p_pl, a_pl in ((pxs, PX, AX), (pys, PY, AY),
                                    (pzs, PZ, AZ)):
                pltpu.sync_copy(a_pl.at[pl.ds(vb, n)], stg.at[pl.ds(0, n)])
                pltpu.sync_copy(stg.at[pl.ds(0, n)], p_pl.at[pl.ds(vb, n)])
        plsc.subcore_barrier()
        return 0
    lax.fori_loop(0, ITERATION, iter_body, 0)

    def final_chunk(vb, n):
        fb = (c * NVp + vb) * 3
        pltpu.sync_copy(AX.at[pl.ds(vb, n)], pxs.at[pl.ds(0, n)])
        pltpu.sync_copy(AY.at[pl.ds(vb, n)], pys.at[pl.ds(0, n)])
        pltpu.sync_copy(AZ.at[pl.ds(vb, n)], pzs.at[pl.ds(0, n)])
        pltpu.sync_copy(VpF.at[pl.ds(fb, n * 3)], sa.at[pl.ds(0, n * 3)])

        def fg(g, _):
            base = 48 * g
            dg = pl.ds(g * 16, 16)
            for col, stg in ((0, pxs), (1, pys), (2, pzs)):
                px = stg[dg]
                xv = plsc.load_gather(sa, [iota3 + (base + col)])
                plsc.store_scatter(sb, [iota3 + (base + col)], px)
                plsc.store_scatter(sc_, [iota3 + (base + col)],
                                   (px - xv) * (1.0 / DT))
            return 0
        lax.fori_loop(0, n // 16, fg, 0)
        pltpu.sync_copy(sb.at[pl.ds(0, n * 3)], VoF.at[pl.ds(fb, n * 3)])
        pltpu.sync_copy(sc_.at[pl.ds(0, n * 3)], VeloF.at[pl.ds(fb, n * 3)])

    def fin_body(i, _):
        final_chunk(v0 + i * 512, 512)
        return 0
    lax.fori_loop(0, VT // 512, fin_body, 0)
    if VT % 512:
        final_chunk(v0 + VT - VT % 512, VT % 512)


def kernel(V, V_velocity, V_w, V_mass, V_force, V_compliance, C_dist, C_init_d):
    B, NV, _ = V.shape
    E = C_dist.shape[0]
    assert B == NC
    VT = _ceil_to(-(-NV // NS), 16)
    NVp = VT * NS
    ET = _ceil_to(-(-E // NS), CHUNK)
    Ep = ET * NS
    NCH = ET // CHUNK

    pv = NVp - NV
    f32 = jnp.float32
    i32 = jnp.int32
    VpF = jnp.concatenate([V, jnp.zeros((B, pv, 3), f32)], axis=1).reshape(-1)
    VelpF = jnp.concatenate([V_velocity, jnp.zeros((B, pv, 3), f32)],
                            axis=1).reshape(-1)
    FpF = jnp.concatenate([V_force, jnp.zeros((B, pv, 3), f32)],
                          axis=1).reshape(-1)
    MWC = jnp.concatenate([V_mass, V_w, V_compliance], axis=-1)
    MWCpF = jnp.concatenate(
        [MWC, jnp.broadcast_to(jnp.array([1.0, 1.0, 0.0], f32), (B, pv, 3))],
        axis=1).reshape(-1)

    pe = Ep - E
    i0f = jnp.concatenate([C_dist[:, 0].astype(i32),
                           jnp.full((pe,), NV, i32)])
    i1f = jnp.concatenate([C_dist[:, 1].astype(i32),
                           jnp.full((pe,), NV, i32)])
    d0f = jnp.concatenate([C_init_d[:, 0], jnp.zeros((pe,), f32)])

    mesh = plsc.VectorSubcoreMesh(core_axis_name="c", subcore_axis_name="s")
    body = functools.partial(_body, NVp, VT, ET, NCH, Ep)
    run = pl.kernel(
        body,
        out_type=(jax.ShapeDtypeStruct((B * NVp * 3,), f32),
                  jax.ShapeDtypeStruct((B * NVp * 3,), f32))
                 + (jax.ShapeDtypeStruct((NC * Ep,), f32),) * 5,
        mesh=mesh,
        compiler_params=pltpu.CompilerParams(use_tc_tiling_on_sc=False,
                                             needs_layout_passes=False),
        scratch_types=(
            [pltpu.VMEM_SHARED((NVp,), f32) for _ in range(8)]
            + ([pltpu.VMEM((CHUNK,), f32),
                pltpu.VMEM((2 * CHUNK,), i32),
                pltpu.VMEM((CHUNK,), f32),
                pltpu.VMEM((CHUNK // 128, 128), i32),
                pltpu.VMEM((CHUNK // 128, 128), i32)]
               + [pltpu.VMEM((CHUNK,), f32) for _ in range(16)]) * 2
            + [pltpu.VMEM((1536,), f32) for _ in range(4)]
            + [pltpu.VMEM((512,), f32) for _ in range(5)]
            + [pltpu.SemaphoreType.DMA for _ in range(3)]
        ),
    )
    VoF, VeloF, *_ = run(VpF, VelpF, FpF, MWCpF, i0f, i1f, d0f)
    Vo = VoF.reshape(B, NVp, 3)[:, :NV, :]
    Velo = VeloF.reshape(B, NVp, 3)[:, :NV, :]
    return Vo, Velo


# speedup vs baseline: 1.3256x; 1.0967x over previous
import functools

import jax
import jax.numpy as jnp
from jax import lax
from jax.experimental import pallas as pl
from jax.experimental.pallas import tpu as pltpu
from jax.experimental.pallas import tpu_sc as plsc

DT = 0.01
ITERATION = 3

NC = 2
NS = 16
CHUNK = 2048
GROUPS = CHUNK // 16


def _ceil_to(x, m):
    return (x + m - 1) // m * m


def _body(NVp, VT, ET, NCH, EPALL,
          VpF, VelpF, FpF, MWCpF, i0f, i1f, d0f,
          VoF, VeloF, Lout, W0e, W1e, Ae_, rDe,
          *scr):
    (PX, PY, PZ, AX, AY, AZ, W, Q) = scr[0:8]
    setA = scr[8:29]
    setB = scr[29:50]
    (sa, sb, sc_, sd, pxs, pys, pzs, ws, qs) = scr[50:59]
    (sem, seml, sems) = scr[59:62]
    c = lax.axis_index("c")
    s = lax.axis_index("s")
    v0 = s * VT
    e0 = s * ET

    iota = lax.iota(jnp.int32, 16)
    iota3 = iota * 3
    MAGIC = jnp.full((16,), 0x5F3759DF, jnp.int32)
    fzero = jnp.zeros((16,), jnp.float32)

    def predict_chunk(vb, n):
        fb = (c * NVp + vb) * 3
        pltpu.sync_copy(VpF.at[pl.ds(fb, n * 3)], sa.at[pl.ds(0, n * 3)])
        pltpu.sync_copy(VelpF.at[pl.ds(fb, n * 3)], sb.at[pl.ds(0, n * 3)])
        pltpu.sync_copy(FpF.at[pl.ds(fb, n * 3)], sc_.at[pl.ds(0, n * 3)])
        pltpu.sync_copy(MWCpF.at[pl.ds(fb, n * 3)], sd.at[pl.ds(0, n * 3)])

        def pg(g, _):
            base = 48 * g
            dg = pl.ds(g * 16, 16)
            mv = plsc.load_gather(sd, [iota3 + base])
            k = (DT * DT) / mv
            ws[dg] = plsc.load_gather(sd, [iota3 + (base + 1)])
            qs[dg] = plsc.load_gather(sd, [iota3 + (base + 2)])
            for col, dst in ((0, pxs), (1, pys), (2, pzs)):
                x = plsc.load_gather(sa, [iota3 + (base + col)])
                v = plsc.load_gather(sb, [iota3 + (base + col)])
                f = plsc.load_gather(sc_, [iota3 + (base + col)])
                dst[dg] = x + DT * v + k * f
            return 0
        lax.fori_loop(0, n // 16, pg, 0)
        for stg, p_pl, a_pl in ((pxs, PX, AX), (pys, PY, AY), (pzs, PZ, AZ)):
            pltpu.sync_copy(stg.at[pl.ds(0, n)], p_pl.at[pl.ds(vb, n)])
            pltpu.sync_copy(stg.at[pl.ds(0, n)], a_pl.at[pl.ds(vb, n)])
        pltpu.sync_copy(ws.at[pl.ds(0, n)], W.at[pl.ds(vb, n)])
        pltpu.sync_copy(qs.at[pl.ds(0, n)], Q.at[pl.ds(vb, n)])

    def pred_body(i, _):
        predict_chunk(v0 + i * 512, 512)
        return 0
    lax.fori_loop(0, VT // 512, pred_body, 0)
    if VT % 512:
        predict_chunk(v0 + VT - VT % 512, VT % 512)
    plsc.subcore_barrier()

    def load_chunk(eb, bufs):
        (lbuf, ibuf, d0c) = bufs[0:3]
        hs = [pltpu.async_copy(i0f.at[pl.ds(e0 + eb, CHUNK)],
                               ibuf.at[pl.ds(0, CHUNK)], seml),
              pltpu.async_copy(i1f.at[pl.ds(e0 + eb, CHUNK)],
                               ibuf.at[pl.ds(CHUNK, CHUNK)], seml),
              pltpu.async_copy(d0f.at[pl.ds(e0 + eb, CHUNK)], d0c, seml),
              pltpu.async_copy(Lout.at[pl.ds(c * EPALL + e0 + eb, CHUNK)],
                               lbuf, seml),
              pltpu.async_copy(W0e.at[pl.ds(c * EPALL + e0 + eb, CHUNK)],
                               bufs[8], seml),
              pltpu.async_copy(W1e.at[pl.ds(c * EPALL + e0 + eb, CHUNK)],
                               bufs[13], seml),
              pltpu.async_copy(Ae_.at[pl.ds(c * EPALL + e0 + eb, CHUNK)],
                               bufs[9], seml),
              pltpu.async_copy(rDe.at[pl.ds(c * EPALL + e0 + eb, CHUNK)],
                               bufs[14], seml)]
        return hs

    def stage_idx(bufs):
        (ibuf, idx0, idx1) = (bufs[1], bufs[3], bufs[4])

        def st(t, _):
            rowi = lax.shift_right_logical(jnp.full((16,), 0, jnp.int32) + t, 3)
            coli = iota + lax.shift_left(
                lax.bitwise_and(t, 7), 4)
            plsc.store_scatter(idx0, [rowi, coli], ibuf[pl.ds(t * 16, 16)])
            plsc.store_scatter(idx1, [rowi, coli],
                               ibuf[pl.ds(CHUNK + t * 16, 16)])
            return 0
        lax.fori_loop(0, GROUPS, st, 0)

    def fire_gathers(bufs):
        (idx0, idx1) = bufs[3:5]
        (x0b, y0b, z0b, w0b, q0b, x1b, y1b, z1b, w1b, q1b) = bufs[5:15]
        hs = []
        for j in range(CHUNK // 128):
            dj = pl.ds(j * 128, 128)
            r0 = idx0.at[j]
            r1 = idx1.at[j]
            for plane, dst in ((PX, x0b), (PY, y0b), (PZ, z0b)):
                hs.append(pltpu.async_copy(plane.at[r0], dst.at[dj], sem))
            for plane, dst in ((PX, x1b), (PY, y1b), (PZ, z1b)):
                hs.append(pltpu.async_copy(plane.at[r1], dst.at[dj], sem))
        return hs

    def compute_chunk(eb, bufs, first):
        (lbuf, d0c) = (bufs[0], bufs[2])
        (x0b, y0b, z0b, w0b, q0b, x1b, y1b, z1b, w1b, q1b) = bufs[5:15]
        (ux0, uy0, uz0, ux1, uy1, uz1) = bufs[15:21]

        def one(dg):
            x0 = x0b[dg]
            y0 = y0b[dg]
            z0 = z0b[dg]
            w0 = w0b[dg]
            q0 = q0b[dg]
            x1 = x1b[dg]
            y1 = y1b[dg]
            z1 = z1b[dg]
            w1 = w1b[dg]
            q1 = q1b[dg]
            dx = x0 - x1
            dy = y0 - y1
            dz = z0 - z1
            d2 = dx * dx + dy * dy + dz * dz
            ib = MAGIC - lax.shift_right_arithmetic(
                plsc.bitcast(d2, jnp.int32), 1)
            r = plsc.bitcast(ib, jnp.float32)
            r = r * (1.5 - 0.5 * d2 * r * r)
            r = r * (1.5 - 0.5 * d2 * r * r)
            r = r * (1.5 - 0.5 * d2 * r * r)
            d0v = d0c[dg]
            A = q0
            rD = q1
            Cc = d2 * r - d0v
            Lv = jnp.where(first, 0.0, lbuf[dg])
            Ld = fzero - (Cc + A * Lv) * rD
            lbuf[dg] = Lv + Ld
            t = Ld * r
            g0 = w0 * t
            g1 = fzero - w1 * t
            ux0[dg] = g0 * dx
            uy0[dg] = g0 * dy
            uz0[dg] = g0 * dz
            ux1[dg] = g1 * dx
            uy1[dg] = g1 * dy
            uz1[dg] = g1 * dz

        def grp(g, _):
            one(pl.ds(g * 32, 16))
            one(pl.ds(g * 32 + 16, 16))
            return 0
        lax.fori_loop(0, GROUPS // 2, grp, 0)
        pltpu.sync_copy(bufs[0], Lout.at[pl.ds(c * EPALL + e0 + eb, CHUNK)])

    def fire_scatters(bufs):
        (idx0, idx1) = bufs[3:5]
        (ux0, uy0, uz0, ux1, uy1, uz1) = bufs[15:21]
        hs = []
        for j in range(CHUNK // 128):
            dj = pl.ds(j * 128, 128)
            r0 = idx0.at[j]
            r1 = idx1.at[j]
            for usrc, plane in ((ux0, AX), (uy0, AY), (uz0, AZ)):
                hs.append(pltpu.async_copy(usrc.at[dj], plane.at[r0],
                                           sems, add=True))
            for usrc, plane in ((ux1, AX), (uy1, AY), (uz1, AZ)):
                hs.append(pltpu.async_copy(usrc.at[dj], plane.at[r1],
                                           sems, add=True))
        return hs

    def pre_body(ci, _):
        eb = ci * CHUNK
        (lbuf, ibuf, d0c, idx0, idx1) = setA[0:5]
        (x0b, y0b, z0b, w0b, q0b, x1b, y1b, z1b, w1b, q1b) = setA[5:15]
        hl = [pltpu.async_copy(i0f.at[pl.ds(e0 + eb, CHUNK)],
                               ibuf.at[pl.ds(0, CHUNK)], seml),
              pltpu.async_copy(i1f.at[pl.ds(e0 + eb, CHUNK)],
                               ibuf.at[pl.ds(CHUNK, CHUNK)], seml)]
        for h in hl:
            h.wait()
        stage_idx(setA)
        hs = []
        for j in range(CHUNK // 128):
            dj = pl.ds(j * 128, 128)
            for plane, dst in ((W, w0b), (Q, q0b)):
                hs.append(pltpu.async_copy(plane.at[idx0.at[j]],
                                           dst.at[dj], sem))
            for plane, dst in ((W, w1b), (Q, q1b)):
                hs.append(pltpu.async_copy(plane.at[idx1.at[j]],
                                           dst.at[dj], sem))
        for h in hs:
            h.wait()

        def pg(g, _):
            dg = pl.ds(g * 16, 16)
            A = 0.5 * (q0b[dg] + q1b[dg])
            rD = 1.0 / (w0b[dg] + w1b[dg] + A)
            x0b[dg] = A
            y0b[dg] = rD
            return 0
        lax.fori_loop(0, GROUPS, pg, 0)
        base = pl.ds(c * EPALL + e0 + eb, CHUNK)
        pltpu.sync_copy(w0b, W0e.at[base])
        pltpu.sync_copy(w1b, W1e.at[base])
        pltpu.sync_copy(x0b, Ae_.at[base])
        pltpu.sync_copy(y0b, rDe.at[base])
        return 0
    lax.fori_loop(0, NCH, pre_body, 0)

    def iter_body(itv, _):
        first = itv == 0

        def pair_body(ci, _):
            ebA = ci * (2 * CHUNK)
            ebB = ebA + CHUNK
            hlA = load_chunk(ebA, setA)
            hlB = load_chunk(ebB, setB)
            for h in hlA:
                h.wait()
            stage_idx(setA)
            hgA = fire_gathers(setA)
            for h in hlB:
                h.wait()
            stage_idx(setB)
            hgB = fire_gathers(setB)
            for h in hgA:
                h.wait()
            compute_chunk(ebA, setA, first)
            hsA = fire_scatters(setA)
            for h in hgB:
                h.wait()
            compute_chunk(ebB, setB, first)
            hsB = fire_scatters(setB)
            for h in hsA:
                h.wait()
            for h in hsB:
                h.wait()
            return 0
        lax.fori_loop(0, NCH // 2, pair_body, 0)
        plsc.subcore_barrier()
        def copy_body(i2, _):
            vb = v0 + i2 * 512
            for stg, p_pl, a_pl in ((pxs, PX, AX), (pys, PY, AY),
                                    (pzs, PZ, AZ)):
                pltpu.sync_copy(a_pl.at[pl.ds(vb, 512)], stg.at[pl.ds(0, 512)])
                pltpu.sync_copy(stg.at[pl.ds(0, 512)], p_pl.at[pl.ds(vb, 512)])
            return 0
        lax.fori_loop(0, VT // 512, copy_body, 0)
        if VT % 512:
            vb = v0 + VT - VT % 512
            n = VT % 512
            for stg, p_pl, a_pl in ((pxs, PX, AX), (pys, PY, AY),
                                    (pzs, PZ, AZ)):
                pltpu.sync_copy(a_pl.at[pl.ds(vb, n)], stg.at[pl.ds(0, n)])
                pltpu.sync_copy(stg.at[pl.ds(0, n)], p_pl.at[pl.ds(vb, n)])
        plsc.subcore_barrier()
        return 0
    lax.fori_loop(0, ITERATION, iter_body, 0)

    def final_chunk(vb, n):
        fb = (c * NVp + vb) * 3
        pltpu.sync_copy(AX.at[pl.ds(vb, n)], pxs.at[pl.ds(0, n)])
        pltpu.sync_copy(AY.at[pl.ds(vb, n)], pys.at[pl.ds(0, n)])
        pltpu.sync_copy(AZ.at[pl.ds(vb, n)], pzs.at[pl.ds(0, n)])
        pltpu.sync_copy(VpF.at[pl.ds(fb, n * 3)], sa.at[pl.ds(0, n * 3)])

        def fg(g, _):
            base = 48 * g
            dg = pl.ds(g * 16, 16)
            for col, stg in ((0, pxs), (1, pys), (2, pzs)):
                px = stg[dg]
                xv = plsc.load_gather(sa, [iota3 + (base + col)])
                plsc.store_scatter(sb, [iota3 + (base + col)], px)
                plsc.store_scatter(sc_, [iota3 + (base + col)],
                                   (px - xv) * (1.0 / DT))
            return 0
        lax.fori_loop(0, n // 16, fg, 0)
        pltpu.sync_copy(sb.at[pl.ds(0, n * 3)], VoF.at[pl.ds(fb, n * 3)])
        pltpu.sync_copy(sc_.at[pl.ds(0, n * 3)], VeloF.at[pl.ds(fb, n * 3)])

    def fin_body(i, _):
        final_chunk(v0 + i * 512, 512)
        return 0
    lax.fori_loop(0, VT // 512, fin_body, 0)
    if VT % 512:
        final_chunk(v0 + VT - VT % 512, VT % 512)


def kernel(V, V_velocity, V_w, V_mass, V_force, V_compliance, C_dist, C_init_d):
    B, NV, _ = V.shape
    E = C_dist.shape[0]
    assert B == NC
    VT = _ceil_to(-(-NV // NS), 16)
    NVp = VT * NS
    ET = _ceil_to(-(-E // NS), CHUNK)
    Ep = ET * NS
    NCH = ET // CHUNK

    pv = NVp - NV
    f32 = jnp.float32
    i32 = jnp.int32
    VpF = jnp.concatenate([V, jnp.zeros((B, pv, 3), f32)], axis=1).reshape(-1)
    VelpF = jnp.concatenate([V_velocity, jnp.zeros((B, pv, 3), f32)],
                            axis=1).reshape(-1)
    FpF = jnp.concatenate([V_force, jnp.zeros((B, pv, 3), f32)],
                          axis=1).reshape(-1)
    MWC = jnp.concatenate([V_mass, V_w, V_compliance], axis=-1)
    MWCpF = jnp.concatenate(
        [MWC, jnp.broadcast_to(jnp.array([1.0, 1.0, 0.0], f32), (B, pv, 3))],
        axis=1).reshape(-1)

    pe = Ep - E
    i0f = jnp.concatenate([C_dist[:, 0].astype(i32),
                           jnp.full((pe,), NV, i32)])
    i1f = jnp.concatenate([C_dist[:, 1].astype(i32),
                           jnp.full((pe,), NV, i32)])
    d0f = jnp.concatenate([C_init_d[:, 0], jnp.zeros((pe,), f32)])

    mesh = plsc.VectorSubcoreMesh(core_axis_name="c", subcore_axis_name="s")
    body = functools.partial(_body, NVp, VT, ET, NCH, Ep)
    run = pl.kernel(
        body,
        out_type=(jax.ShapeDtypeStruct((B * NVp * 3,), f32),
                  jax.ShapeDtypeStruct((B * NVp * 3,), f32))
                 + (jax.ShapeDtypeStruct((NC * Ep,), f32),) * 5,
        mesh=mesh,
        compiler_params=pltpu.CompilerParams(use_tc_tiling_on_sc=False,
                                             needs_layout_passes=False),
        scratch_types=(
            [pltpu.VMEM_SHARED((NVp,), f32) for _ in range(8)]
            + ([pltpu.VMEM((CHUNK,), f32),
                pltpu.VMEM((2 * CHUNK,), i32),
                pltpu.VMEM((CHUNK,), f32),
                pltpu.VMEM((CHUNK // 128, 128), i32),
                pltpu.VMEM((CHUNK // 128, 128), i32)]
               + [pltpu.VMEM((CHUNK,), f32) for _ in range(16)]) * 2
            + [pltpu.VMEM((1536,), f32) for _ in range(4)]
            + [pltpu.VMEM((512,), f32) for _ in range(5)]
            + [pltpu.SemaphoreType.DMA for _ in range(3)]
        ),
    )
    VoF, VeloF, *_ = run(VpF, VelpF, FpF, MWCpF, i0f, i1f, d0f)
    Vo = VoF.reshape(B, NVp, 3)[:, :NV, :]
    Velo = VeloF.reshape(B, NVp, 3)[:, :NV, :]
    return Vo, Velo
